# Initial kernel scaffold; baseline (speedup 1.0000x reference)
import jax
import jax.numpy as jnp
from jax.experimental import pallas as pl


def kernel(x, Wq, Wk, Wv, Wproj, cq_w, cq_b, ck_w, ck_b, cv_w, cv_b, ga_w, ga_b, ge_w, ge_b, gt_w, gt_b, gg_w, gg_b, poly_coeffs, ln_gamma, rg_w):
    raise NotImplementedError("write your pallas kernel here")


# speedup vs baseline: 9.8137x; 9.8137x over previous
import functools
import math

import jax
import jax.numpy as jnp
from jax.experimental import pallas as pl
from jax.experimental.pallas import tpu as pltpu

B, T, C = 2, 1024, 1024
H, D = 8, 64
DI = H * D
CS = 64
NC = T // CS
NS_STEPS = 3
OMEGA = 16
MAX_LR = 0.1
K = 4

PE_COEFFS = [(8.156554524902461, -22.48329292557795, 15.878769915207462),
             (4.042929935166739, -2.808917465908714, 0.5000178451051316),
             (3.8916678022926607, -2.772484153217685, 0.5060648178503393)]

_HI = jax.lax.Precision.HIGHEST


def _dot3(a, b, dn):
    f32 = jnp.float32
    ah = a.astype(jnp.bfloat16)
    al = (a - ah.astype(f32)).astype(jnp.bfloat16)
    bh = b.astype(jnp.bfloat16)
    bl = (b - bh.astype(f32)).astype(jnp.bfloat16)
    r = jax.lax.dot_general(ah, bl, dn, preferred_element_type=f32)
    r = r + jax.lax.dot_general(al, bh, dn, preferred_element_type=f32)
    r = r + jax.lax.dot_general(ah, bh, dn, preferred_element_type=f32)
    return r


def _dot1(a, b, dn):
    return jax.lax.dot_general(a, b, dn, preferred_element_type=jnp.float32,
                               precision=jax.lax.Precision.DEFAULT)


_DOTMODE = {'qkv': '3x', 'gate': '3x', 'pred': '3x', 'polar': '3x',
            'scan': '3x', 'out': '3x'}


def _sdot(site, a, b, dn):
    return (_dot3 if _DOTMODE[site] == '3x' else _dot1)(a, b, dn)


def _log_sigmoid(z):
    return jnp.where(z >= 0, -jnp.log1p(jnp.exp(-z)), z - jnp.log1p(jnp.exp(z)))


def _proj_kernel(x_ref, wq_ref, wk_ref, wv_ref, cq_ref, ck_ref, cv_ref,
                 cb_ref, gw_ref, gb_ref, pc_ref,
                 q_ref, kphi_ref, v_ref, g_ref):
    xb = x_ref[0]
    dn = (((1,), (1,)), ((), ()))

    def conv(xw, cw_ref, bias_row):
        acc = cw_ref[K - 1:K, :] * xw
        for j in range(K - 1):
            shift = K - 1 - j
            sh = jnp.concatenate(
                [jnp.zeros((shift, DI), jnp.float32), xw[:T - shift, :]], axis=0)
            acc = acc + cw_ref[j:j + 1, :] * sh
        return acc + bias_row

    xq = _sdot('qkv', xb, wq_ref[...], dn)
    q_ref[0] = conv(xq, cq_ref, cb_ref[0:1, :])
    xv = _sdot('qkv', xb, wv_ref[...], dn)
    v_ref[0] = conv(xv, cv_ref, cb_ref[2:3, :])
    xk = _sdot('qkv', xb, wk_ref[...], dn)
    k = conv(xk, ck_ref, cb_ref[1:2, :])
    kphi_ref[0] = pc_ref[0:1, 0:1] * k + pc_ref[0:1, 1:2] * (k * k)

    glog = _sdot('gate', xb, gw_ref[...], dn)
    g_ref[0] = glog + gb_ref[...]


def _tri_masks():
    r = jax.lax.broadcasted_iota(jnp.int32, (CS, CS), 0)
    c = jax.lax.broadcasted_iota(jnp.int32, (CS, CS), 1)
    tril = (c <= r)
    band = tril & (c > r - OMEGA)
    trif = jnp.where(tril, 1.0, 0.0).astype(jnp.float32)
    bandf = jnp.where(band, 1.0, 0.0).astype(jnp.float32)
    return trif, bandf, tril


def _scan_weights(z_row, z_col, trif, tril):
    lr = _log_sigmoid(z_row)
    lc = _log_sigmoid(z_col)
    dncol = (((1,), (0,)), ((), ()))
    lcum_c = jax.lax.dot_general(trif, lc, dncol, precision=_HI)
    dnrow = (((1,), (1,)), ((), ()))
    lcum_r = jax.lax.dot_general(lr, trif, dnrow, precision=_HI)
    w = jnp.where(tril, jnp.exp(lcum_c - lcum_r), 0.0)
    p = jnp.exp(lcum_c)
    return w, p


def _polar_express(y):
    fn = jnp.sqrt(jnp.sum(y * y, axis=(1, 2), keepdims=True) + 1e-12)
    x = y / (fn * 1.01 + 1e-6)
    dn_gram = (((1,), (1,)), ((0,), (0,)))
    dn_mm = (((2,), (1,)), ((0,), (0,)))
    for a, b, c in PE_COEFFS[:NS_STEPS]:
        bm = _sdot('polar', x, x, dn_gram)
        b2 = _sdot('polar', bm, bm, dn_mm)
        p = b * bm + c * b2
        x = a * x + _sdot('polar', x, p, dn_mm)
    return x


def _rec_kernel(q_ref, kphi_ref, v_ref, g_ref, lng_ref, y_ref):
    trif, bandf, tril = _tri_masks()
    lng_row = lng_ref[0]
    dn_scan = (((1,), (0,)), ((), ()))
    dn_mm2 = (((1,), (0,)), ((), ()))

    def chunk(cidx, carry):
        mt, st = carry
        t0 = cidx * CS
        q_c = q_ref[0, 0, pl.ds(t0, CS), :]
        kphi_c = kphi_ref[0, 0, pl.ds(t0, CS), :]
        v_c = v_ref[0, 0, pl.ds(t0, CS), :]
        gct = g_ref[0, 0, pl.ds(t0, CS), :]
        gc = jnp.transpose(gct)

        pred = _sdot('pred', kphi_c, mt, dn_mm2)
        err = pred - v_c
        u3 = kphi_c[:, :, None] * err[:, None, :]

        eta = MAX_LR * jax.nn.sigmoid(gct[:, 1:2])
        gam_row = jax.nn.sigmoid(gc[3:4, :])
        wc = (-2.0 * eta) * bandf * gam_row
        wth, pth = _scan_weights(gc[2:3, :], gct[:, 2:3], trif, tril)
        ws = jax.lax.dot_general(wth, wc, dn_mm2, precision=_HI)
        s_all = (_sdot('scan', ws, u3, dn_scan)
                 + pth[:, :, None] * st[None])
        s_fin = s_all[CS - 1]

        sp = _polar_express(s_all)

        wal, pal = _scan_weights(gc[0:1, :], gct[:, 0:1], trif, tril)
        m_all = (_sdot('scan', wal, sp, dn_scan)
                 + pal[:, :, None] * mt[None])
        m_fin = m_all[CS - 1]

        y = jnp.sum(m_all * q_c[:, :, None], axis=1)

        ms = jnp.mean(y * y, axis=-1, keepdims=True)
        yn = y * jax.lax.rsqrt(ms + 1e-6) * (1.0 + lng_row)
        yn = yn * jax.nn.sigmoid(gct[:, 4:5])
        y_ref[0, 0, pl.ds(t0, CS), :] = yn
        return m_fin, s_fin

    z = jnp.zeros((D, D), jnp.float32)
    jax.lax.fori_loop(0, NC, chunk, (z, z), unroll=False)


def _out_kernel(y_ref, wp_ref, o_ref):
    dn = (((1,), (1,)), ((), ()))
    yflat = jnp.concatenate([y_ref[0, h] for h in range(H)], axis=1)
    o_ref[0] = _sdot('out', yflat, wp_ref[...], dn)


def _stage1(x, Wq, Wk, Wv, cq_w, cq_b, ck_w, ck_b, cv_w, cv_b,
            ga_w, ga_b, ge_w, ge_b, gt_w, gt_b, gg_w, gg_b,
            poly_coeffs, rg_w):
    f = jnp.float32
    cmp = pltpu.CompilerParams

    cq = jnp.transpose(cq_w.reshape(DI, K))
    ck = jnp.transpose(ck_w.reshape(DI, K))
    cv = jnp.transpose(cv_w.reshape(DI, K))
    cb = jnp.stack([cq_b, ck_b, cv_b], axis=0)

    zw = jnp.zeros_like(ga_w)
    gw = jnp.stack([ga_w, ge_w, gt_w, gg_w, rg_w, zw, zw, zw], axis=1)
    gw = gw.reshape(8 * H, C)
    zb = jnp.zeros_like(ga_b)
    gb = jnp.stack([ga_b, ge_b, gt_b, gg_b, zb, zb, zb, zb], axis=1)
    gb = gb.reshape(1, 8 * H)
    pc2 = poly_coeffs.reshape(1, 2)

    q, kphi, v, gates = pl.pallas_call(
        _proj_kernel,
        grid=(B,),
        in_specs=[
            pl.BlockSpec((1, T, C), lambda b: (b, 0, 0)),
            pl.BlockSpec((DI, C), lambda b: (0, 0)),
            pl.BlockSpec((DI, C), lambda b: (0, 0)),
            pl.BlockSpec((DI, C), lambda b: (0, 0)),
            pl.BlockSpec((K, DI), lambda b: (0, 0)),
            pl.BlockSpec((K, DI), lambda b: (0, 0)),
            pl.BlockSpec((K, DI), lambda b: (0, 0)),
            pl.BlockSpec((3, DI), lambda b: (0, 0)),
            pl.BlockSpec((8 * H, C), lambda b: (0, 0)),
            pl.BlockSpec((1, 8 * H), lambda b: (0, 0)),
            pl.BlockSpec((1, 2), lambda b: (0, 0)),
        ],
        out_specs=[
            pl.BlockSpec((1, T, DI), lambda b: (b, 0, 0)),
            pl.BlockSpec((1, T, DI), lambda b: (b, 0, 0)),
            pl.BlockSpec((1, T, DI), lambda b: (b, 0, 0)),
            pl.BlockSpec((1, T, 8 * H), lambda b: (b, 0, 0)),
        ],
        out_shape=[
            jax.ShapeDtypeStruct((B, T, DI), f),
            jax.ShapeDtypeStruct((B, T, DI), f),
            jax.ShapeDtypeStruct((B, T, DI), f),
            jax.ShapeDtypeStruct((B, T, 8 * H), f),
        ],
        compiler_params=cmp(dimension_semantics=("parallel",),
                            vmem_limit_bytes=56 * 1024 * 1024),
    )(x, Wq, Wk, Wv, cq, ck, cv, cb, gw, gb, pc2)
    return q, kphi, v, gates


@jax.jit
def kernel(x, Wq, Wk, Wv, Wproj, cq_w, cq_b, ck_w, ck_b, cv_w, cv_b,
           ga_w, ga_b, ge_w, ge_b, gt_w, gt_b, gg_w, gg_b,
           poly_coeffs, ln_gamma, rg_w):
    f = jnp.float32
    cmp = pltpu.CompilerParams
    q, kphi, v, gates = _stage1(x, Wq, Wk, Wv, cq_w, cq_b, ck_w, ck_b,
                                cv_w, cv_b, ga_w, ga_b, ge_w, ge_b,
                                gt_w, gt_b, gg_w, gg_b, poly_coeffs, rg_w)

    q = q.reshape(B, T, H, D).transpose(0, 2, 1, 3)
    kphi = kphi.reshape(B, T, H, D).transpose(0, 2, 1, 3)
    v = v.reshape(B, T, H, D).transpose(0, 2, 1, 3)
    gates = gates.reshape(B, T, H, 8).transpose(0, 2, 1, 3)

    lng = ln_gamma.reshape(H, 1, D)
    yn = pl.pallas_call(
        _rec_kernel,
        grid=(B * H,),
        in_specs=[
            pl.BlockSpec((1, 1, T, D), lambda i: (i // H, i % H, 0, 0)),
            pl.BlockSpec((1, 1, T, D), lambda i: (i // H, i % H, 0, 0)),
            pl.BlockSpec((1, 1, T, D), lambda i: (i // H, i % H, 0, 0)),
            pl.BlockSpec((1, 1, T, 8), lambda i: (i // H, i % H, 0, 0)),
            pl.BlockSpec((1, 1, D), lambda i: (i % H, 0, 0)),
        ],
        out_specs=pl.BlockSpec((1, 1, T, D), lambda i: (i // H, i % H, 0, 0)),
        out_shape=jax.ShapeDtypeStruct((B, H, T, D), f),
        compiler_params=cmp(dimension_semantics=("parallel",),
                            vmem_limit_bytes=56 * 1024 * 1024),
    )(q, kphi, v, gates, lng)

    out = pl.pallas_call(
        _out_kernel,
        grid=(B,),
        in_specs=[
            pl.BlockSpec((1, H, T, D), lambda b: (b, 0, 0, 0)),
            pl.BlockSpec((C, DI), lambda b: (0, 0)),
        ],
        out_specs=pl.BlockSpec((1, T, C), lambda b: (b, 0, 0)),
        out_shape=jax.ShapeDtypeStruct((B, T, C), f),
        compiler_params=cmp(dimension_semantics=("parallel",),
                            vmem_limit_bytes=56 * 1024 * 1024),
    )(yn, Wproj)
    return out
